# Initial kernel scaffold; baseline (speedup 1.0000x reference)
import jax
import jax.numpy as jnp
from jax.experimental import pallas as pl


def kernel(input, embeddings):
    raise NotImplementedError("write your pallas kernel here")


# speedup vs baseline: 1.1510x; 1.1510x over previous
import jax
import jax.numpy as jnp
from jax import lax
from jax.experimental import pallas as pl
from jax.experimental.pallas import tpu as pltpu
from jax.experimental.pallas import tpu_sc as plsc

D = 32
N = 16384 * 50
NC, NS = 2, 16
NW = NC * NS
PER_W = N // NW
CHUNK = 1600
NCHUNK = PER_W // CHUNK


def _gather_body(idx_hbm, table_hbm, out_hbm, idx_v, rows_v, sem):
    wid = lax.axis_index("s") * NC + lax.axis_index("c")
    base = wid * PER_W

    def chunk(c, carry):
        off = base + c * CHUNK
        pltpu.sync_copy(idx_hbm.at[pl.ds(off, CHUNK)], idx_v)
        pltpu.async_copy(table_hbm.at[idx_v], rows_v, sem).wait()
        pltpu.sync_copy(rows_v, out_hbm.at[pl.ds(off, CHUNK)])
        return carry

    lax.fori_loop(0, NCHUNK, chunk, 0)


@jax.jit
def _gather(idx_flat, table):
    f = pl.kernel(
        _gather_body,
        out_type=jax.ShapeDtypeStruct((N, D), jnp.float32),
        mesh=plsc.VectorSubcoreMesh(core_axis_name="c", subcore_axis_name="s"),
        scratch_types=[
            pltpu.VMEM((CHUNK,), jnp.int32),
            pltpu.VMEM((CHUNK, D), jnp.float32),
            pltpu.SemaphoreType.DMA,
        ],
        compiler_params=pltpu.CompilerParams(use_tc_tiling_on_sc=False),
    )
    return f(idx_flat, table)


def kernel(input, embeddings):
    idx = input.astype(jnp.int32).reshape(-1)
    out = _gather(idx, embeddings)
    return out.reshape(input.shape + (D,))


# speedup vs baseline: 1.1621x; 1.0096x over previous
import jax
import jax.numpy as jnp
from jax import lax
from jax.experimental import pallas as pl
from jax.experimental.pallas import tpu as pltpu
from jax.experimental.pallas import tpu_sc as plsc

D = 32
N = 16384 * 50
NC, NS = 2, 16
NW = NC * NS
PER_W = N // NW
CHUNK = 1600
NCHUNK = PER_W // CHUNK


def _gather_body(idx_hbm, table_hbm, out_hbm, idx_v, rows_v,
                 gsem0, gsem1, ssem0, ssem1):
    wid = lax.axis_index("s") * NC + lax.axis_index("c")
    base = wid * PER_W
    gsem = (gsem0, gsem1)
    ssem = (ssem0, ssem1)

    pltpu.sync_copy(idx_hbm.at[pl.ds(wid * NCHUNK, NCHUNK)], idx_v)

    stores = [None, None]
    gathers = [None, None]
    for c in range(NCHUNK):
        s = c % 2
        if stores[s] is not None:
            stores[s].wait()
        gathers[s] = pltpu.make_async_copy(
            table_hbm.at[idx_v.at[c]], rows_v.at[s], gsem[s])
        gathers[s].start()
        if c > 0:
            p = (c - 1) % 2
            gathers[p].wait()
            stores[p] = pltpu.make_async_copy(
                rows_v.at[p], out_hbm.at[pl.ds(base + (c - 1) * CHUNK, CHUNK)],
                ssem[p])
            stores[p].start()
    last = (NCHUNK - 1) % 2
    gathers[last].wait()
    stores[last] = pltpu.make_async_copy(
        rows_v.at[last],
        out_hbm.at[pl.ds(base + (NCHUNK - 1) * CHUNK, CHUNK)], ssem[last])
    stores[last].start()
    stores[1 - last].wait()
    stores[last].wait()


@jax.jit
def _gather(idx_flat, table):
    f = pl.kernel(
        _gather_body,
        out_type=jax.ShapeDtypeStruct((N, D), jnp.float32),
        mesh=plsc.VectorSubcoreMesh(core_axis_name="c", subcore_axis_name="s"),
        scratch_types=[
            pltpu.VMEM((NCHUNK, CHUNK), jnp.int32),
            pltpu.VMEM((2, CHUNK, D), jnp.float32),
            pltpu.SemaphoreType.DMA,
            pltpu.SemaphoreType.DMA,
            pltpu.SemaphoreType.DMA,
            pltpu.SemaphoreType.DMA,
        ],
        compiler_params=pltpu.CompilerParams(use_tc_tiling_on_sc=False),
    )
    return f(idx_flat, table)


def kernel(input, embeddings):
    idx = input.astype(jnp.int32).reshape(NW * NCHUNK, CHUNK)
    out = _gather(idx, embeddings)
    return out.reshape(input.shape + (D,))


# speedup vs baseline: 2.0279x; 1.7451x over previous
import jax
import jax.numpy as jnp
from jax import lax
from jax.experimental import pallas as pl
from jax.experimental.pallas import tpu as pltpu
from jax.experimental.pallas import tpu_sc as plsc

D = 32
N = 16384 * 50
NC, NS = 2, 16
NW = NC * NS
PER_W = N // NW
CHUNK = 1600
NCHUNK = PER_W // CHUNK


def _gather_body(idx_hbm, table_hbm, out_hbm, idx_v, rows_v,
                 gsem0, gsem1, ssem0, ssem1):
    wid = lax.axis_index("s") * NC + lax.axis_index("c")
    base = wid * PER_W
    gsem = (gsem0, gsem1)
    ssem = (ssem0, ssem1)

    pltpu.sync_copy(idx_hbm.at[pl.ds(base, PER_W)], idx_v)

    stores = [None, None]
    gathers = [None, None]
    for c in range(NCHUNK):
        s = c % 2
        if stores[s] is not None:
            stores[s].wait()
        gathers[s] = pltpu.make_async_copy(
            table_hbm.at[idx_v.at[pl.ds(c * CHUNK, CHUNK)]], rows_v.at[s],
            gsem[s])
        gathers[s].start()
        if c > 0:
            p = (c - 1) % 2
            gathers[p].wait()
            stores[p] = pltpu.make_async_copy(
                rows_v.at[p], out_hbm.at[pl.ds(base + (c - 1) * CHUNK, CHUNK)],
                ssem[p])
            stores[p].start()
    last = (NCHUNK - 1) % 2
    gathers[last].wait()
    stores[last] = pltpu.make_async_copy(
        rows_v.at[last],
        out_hbm.at[pl.ds(base + (NCHUNK - 1) * CHUNK, CHUNK)], ssem[last])
    stores[last].start()
    stores[1 - last].wait()
    stores[last].wait()


@jax.jit
def _gather(idx_flat, table):
    f = pl.kernel(
        _gather_body,
        out_type=jax.ShapeDtypeStruct((N, D), jnp.float32),
        mesh=plsc.VectorSubcoreMesh(core_axis_name="c", subcore_axis_name="s"),
        scratch_types=[
            pltpu.VMEM((PER_W,), jnp.int32),
            pltpu.VMEM((2, CHUNK, D), jnp.float32),
            pltpu.SemaphoreType.DMA,
            pltpu.SemaphoreType.DMA,
            pltpu.SemaphoreType.DMA,
            pltpu.SemaphoreType.DMA,
        ],
        compiler_params=pltpu.CompilerParams(use_tc_tiling_on_sc=False),
    )
    return f(idx_flat, table)


def kernel(input, embeddings):
    b, h = input.shape
    idx = input.T.astype(jnp.int32).reshape(-1)
    out = _gather(idx, embeddings)
    return out.reshape(h, b, D).transpose(1, 0, 2)
